# baseline (speedup 1.0000x reference)
import functools
import jax
import jax.numpy as jnp
from jax import lax
from jax.experimental import pallas as pl
from jax.experimental.pallas import tpu as pltpu
from jax.experimental.pallas import tpu_sc as plsc

N, M, D = 10000, 32, 128
L = 16
NW = 32
NDC = D // L

S_SC = 2560
NPW = S_SC // NW
K = 8
NCH = NPW // K
ROWS = K * M

NB = 80
TC_OFF = S_SC // NB


def _sc_body(x_hbm, o_hbm, buf0, buf1, ob, sem0, sem1):
    c = lax.axis_index("c")
    s = lax.axis_index("s")
    wid = s * 2 + c
    base_node = wid * NPW
    base_row = base_node * M

    def copy_in(g, buf, sem):
        return pltpu.make_async_copy(
            x_hbm.at[pl.ds(base_row + g * ROWS, ROWS)], buf, sem)

    sls = [pl.ds(dc * L, L) for dc in range(NDC)]

    def reduce_group(buf, k0, nk):
        def mstep(m, accs):
            return tuple(
                accs[j * NDC + dc] + buf[(k0 + j) * M + m, sls[dc]]
                for j in range(nk) for dc in range(NDC))

        init = tuple(
            buf[(k0 + j) * M, sls[dc]]
            for j in range(nk) for dc in range(NDC))
        accs = lax.fori_loop(1, M, mstep, init)
        for j in range(nk):
            for dc in range(NDC):
                ob[k0 + j, sls[dc]] = accs[j * NDC + dc]

    def reduce_chunk(buf, g):
        k0 = 0
        while k0 < K:
            nk = min(4, K - k0)
            reduce_group(buf, k0, nk)
            k0 += nk
        pltpu.sync_copy(ob, o_hbm.at[pl.ds(base_node + g * K, K)])

    copy_in(0, buf0, sem0).start()
    copy_in(1, buf1, sem1).start()

    def step(i, carry):
        for b, (buf, sem) in enumerate(((buf0, sem0), (buf1, sem1))):
            g = i * 2 + b
            copy_in(g, buf, sem).wait()
            reduce_chunk(buf, g)
            nxt = g + 2

            @pl.when(nxt < NCH)
            def _(buf=buf, sem=sem, nxt=nxt):
                copy_in(nxt, buf, sem).start()
        return carry

    lax.fori_loop(0, NCH // 2, step, 0)


_sc_call = functools.partial(
    pl.kernel,
    out_type=jax.ShapeDtypeStruct((S_SC, D), jnp.float32),
    mesh=plsc.VectorSubcoreMesh(core_axis_name="c", subcore_axis_name="s"),
    scratch_types=[
        pltpu.VMEM((ROWS, D), jnp.float32),
        pltpu.VMEM((ROWS, D), jnp.float32),
        pltpu.VMEM((K, D), jnp.float32),
        pltpu.SemaphoreType.DMA,
        pltpu.SemaphoreType.DMA,
    ],
)(_sc_body)


def _tc_body(x_ref, o_ref):
    o_ref[...] = jnp.sum(x_ref[...], axis=1)


def _tc_call(x3):
    return pl.pallas_call(
        _tc_body,
        grid=((N - S_SC) // NB,),
        in_specs=[pl.BlockSpec((NB, M, D), lambda i: (i + TC_OFF, 0, 0))],
        out_specs=pl.BlockSpec((NB, D), lambda i: (i + TC_OFF, 0)),
        out_shape=jax.ShapeDtypeStruct((N, D), jnp.float32),
    )(x3)


def kernel(messages):
    x3 = messages.reshape(N, M, D)
    sc_out = _sc_call(x3.reshape(N * M, D))
    tc_full = _tc_call(x3)
    out = lax.dynamic_update_slice(tc_full, sc_out, (0, 0))
    return out.reshape(1, N, D)


# speedup vs baseline: 1.3592x; 1.3592x over previous
import functools
import jax
import jax.numpy as jnp
from jax import lax
from jax.experimental import pallas as pl
from jax.experimental.pallas import tpu as pltpu
from jax.experimental.pallas import tpu_sc as plsc

N, M, D = 10000, 32, 128
L = 16
NW = 32
NDC = D // L

S_SC = 4000
K = 8
ROWS = K * M
CH = S_SC // K
CH_Q, CH_R = CH // NW, CH % NW

NB = 400
TC_OFF = S_SC // NB


def _sc_body(x_hbm, o_hbm, buf0, buf1, ob, sem0, sem1):
    c = lax.axis_index("c")
    s = lax.axis_index("s")
    wid = s * 2 + c
    ncw = CH_Q + jnp.where(wid < CH_R, 1, 0)

    def copy_in(g, buf, sem):
        row = (wid + g * NW) * ROWS
        return pltpu.make_async_copy(x_hbm.at[pl.ds(row, ROWS)], buf, sem)

    sls = [pl.ds(dc * L, L) for dc in range(NDC)]

    def reduce_group(buf, k0, nk):
        def mstep(m, accs):
            return tuple(
                accs[j * NDC + dc] + buf[(k0 + j) * M + m, sls[dc]]
                for j in range(nk) for dc in range(NDC))

        init = tuple(
            buf[(k0 + j) * M, sls[dc]]
            for j in range(nk) for dc in range(NDC))
        accs = lax.fori_loop(1, M, mstep, init)
        for j in range(nk):
            for dc in range(NDC):
                ob[k0 + j, sls[dc]] = accs[j * NDC + dc]

    def reduce_chunk(buf, g):
        for k0 in range(0, K, 4):
            reduce_group(buf, k0, min(4, K - k0))
        pltpu.sync_copy(ob, o_hbm.at[pl.ds((wid + g * NW) * K, K)])

    copy_in(0, buf0, sem0).start()
    copy_in(1, buf1, sem1).start()

    def step(i, carry):
        for b, (buf, sem) in enumerate(((buf0, sem0), (buf1, sem1))):
            g = i * 2 + b

            @pl.when(g < ncw)
            def _(buf=buf, sem=sem, g=g):
                copy_in(g, buf, sem).wait()
                reduce_chunk(buf, g)

                @pl.when(g + 2 < ncw)
                def _():
                    copy_in(g + 2, buf, sem).start()
        return carry

    lax.fori_loop(0, (ncw + 1) // 2, step, 0)


_sc_call = functools.partial(
    pl.kernel,
    out_type=jax.ShapeDtypeStruct((S_SC, D), jnp.float32),
    mesh=plsc.VectorSubcoreMesh(core_axis_name="c", subcore_axis_name="s"),
    scratch_types=[
        pltpu.VMEM((ROWS, D), jnp.float32),
        pltpu.VMEM((ROWS, D), jnp.float32),
        pltpu.VMEM((K, D), jnp.float32),
        pltpu.SemaphoreType.DMA,
        pltpu.SemaphoreType.DMA,
    ],
)(_sc_body)


def _tc_body(x_ref, o_ref):
    o_ref[...] = jnp.sum(x_ref[...], axis=1)


def _tc_call(x3):
    return pl.pallas_call(
        _tc_body,
        grid=((N - S_SC) // NB,),
        in_specs=[pl.BlockSpec((NB, M, D), lambda i: (i + TC_OFF, 0, 0))],
        out_specs=pl.BlockSpec((NB, D), lambda i: (i + TC_OFF, 0)),
        out_shape=jax.ShapeDtypeStruct((N, D), jnp.float32),
    )(x3)


def kernel(messages):
    x3 = messages.reshape(N, M, D)
    sc_out = _sc_call(x3.reshape(N * M, D))
    tc_full = _tc_call(x3)
    out = lax.dynamic_update_slice(tc_full, sc_out, (0, 0))
    return out.reshape(1, N, D)
